# baseline (speedup 1.0000x reference)
import functools

import jax
import jax.numpy as jnp
from jax import lax
from jax.experimental import pallas as pl
from jax.experimental.pallas import tpu as pltpu
from jax.experimental.pallas import tpu_sc as plsc

N = 16384
D = 256
M = 8
NCODES = 256
DSUB = D // M
BN = 8192

_DN = (((1,), (1,)), ((), ()))

NW = 32
RPW = N // NW
LANES = 16


def _pdt_tc_body(x_ref, w_ref, cbn_ref, sumd_ref, h_ref, gidx_ref):
    xb = x_ref[:]
    xb16 = xb.astype(jnp.bfloat16)
    xsq16 = xb16 * xb16
    ones8 = jnp.ones((8, D), jnp.bfloat16)
    xnorm = lax.dot_general(ones8, xsq16, _DN,
                            preferred_element_type=jnp.float32)[0]
    iota_col = lax.broadcasted_iota(
        jnp.int32, (NCODES, 1), 0).astype(jnp.bfloat16)
    big16 = jnp.bfloat16(256.0)
    sum_s = jnp.zeros((BN,), jnp.float32)
    for m in range(M):
        xc16 = xb16[:, m * DSUB:(m + 1) * DSUB]
        cbn_col = cbn_ref[m][:, None]
        s = cbn_col + lax.dot_general(
            w_ref[m], xc16, _DN,
            preferred_element_type=jnp.float32,
        ).astype(jnp.bfloat16)
        smin = jnp.min(s, axis=0)
        mask = s == smin[None, :]
        idx16 = jnp.min(jnp.where(mask, iota_col, big16), axis=0)
        gidx_ref[m, :] = idx16.astype(jnp.int32) + m * NCODES
        sum_s = sum_s + smin.astype(jnp.float32)
    sumd_ref[:] = jnp.maximum(xnorm + sum_s, 0.0)
    h_ref[:] = 0.5 * sum_s + xnorm


def _pdt_sc_body(gidx_hbm, sumd_hbm, h_hbm, cbnf_hbm, out_hbm,
                 gidx_v, sumd_v, h_v, cbnf_v, out_v):
    wid = lax.axis_index("s") * 2 + lax.axis_index("c")
    base = wid * RPW
    pltpu.sync_copy(cbnf_hbm, cbnf_v)
    pltpu.sync_copy(gidx_hbm.at[:, pl.ds(base, RPW)], gidx_v)
    pltpu.sync_copy(sumd_hbm.at[pl.ds(base, RPW)], sumd_v)
    pltpu.sync_copy(h_hbm.at[pl.ds(base, RPW)], h_v)
    half = jnp.full((LANES,), 0.5, jnp.float32)
    tiny = jnp.full((LANES,), 1e-30, jnp.float32)
    magic = jnp.full((LANES,), 0x1FBD1DF5, jnp.int32)
    for i in range(RPW // LANES):
        sl = pl.ds(i * LANES, LANES)
        g = jnp.zeros((LANES,), jnp.float32)
        for m in range(M):
            g = g + plsc.load_gather(cbnf_v, [gidx_v[m, sl]])
        d = jnp.maximum(sumd_v[sl], tiny)
        y = plsc.bitcast((plsc.bitcast(d, jnp.int32) >> 1) + magic,
                         jnp.float32)
        y = half * (y + d / y)
        y = half * (y + d / y)
        y = half * (y + d / y)
        out_v[sl] = y + jnp.abs(half * g - h_v[sl])
    pltpu.sync_copy(out_v, out_hbm.at[pl.ds(base, RPW)])


def kernel(x, codebook):
    cbn_f32 = jnp.sum(codebook * codebook, axis=-1)
    cbn16 = cbn_f32.astype(jnp.bfloat16)
    w = (-2.0 * codebook).astype(jnp.bfloat16)
    cbnf = cbn_f32.reshape(M * NCODES)

    sumd, h, gidx = pl.pallas_call(
        _pdt_tc_body,
        grid=(N // BN,),
        in_specs=[
            pl.BlockSpec((BN, D), lambda i: (i, 0)),
            pl.BlockSpec((M, NCODES, DSUB), lambda i: (0, 0, 0)),
            pl.BlockSpec((M, NCODES), lambda i: (0, 0)),
        ],
        out_specs=[
            pl.BlockSpec((BN,), lambda i: (i,)),
            pl.BlockSpec((BN,), lambda i: (i,)),
            pl.BlockSpec((M, BN), lambda i: (0, i)),
        ],
        out_shape=[
            jax.ShapeDtypeStruct((N,), jnp.float32),
            jax.ShapeDtypeStruct((N,), jnp.float32),
            jax.ShapeDtypeStruct((M, N), jnp.int32),
        ],
    )(x, w, cbn16)

    mesh = plsc.VectorSubcoreMesh(core_axis_name="c", subcore_axis_name="s")
    sc_stage = functools.partial(
        pl.kernel,
        mesh=mesh,
        compiler_params=pltpu.CompilerParams(needs_layout_passes=False),
        out_type=jax.ShapeDtypeStruct((N,), jnp.float32),
        scratch_types=[
            pltpu.VMEM((M, RPW), jnp.int32),
            pltpu.VMEM((RPW,), jnp.float32),
            pltpu.VMEM((RPW,), jnp.float32),
            pltpu.VMEM((M * NCODES,), jnp.float32),
            pltpu.VMEM((RPW,), jnp.float32),
        ],
    )(_pdt_sc_body)
    return sc_stage(gidx, sumd, h, cbnf)


# speedup vs baseline: 1.0408x; 1.0408x over previous
import functools

import jax
import jax.numpy as jnp
from jax import lax
from jax.experimental import pallas as pl
from jax.experimental.pallas import tpu as pltpu
from jax.experimental.pallas import tpu_sc as plsc

N = 16384
D = 256
M = 8
NCODES = 256
DSUB = D // M
BN = 4096

_DN = (((1,), (1,)), ((), ()))

NW = 32
RPW = N // NW
LANES = 16


def _pdt_tc_body(x_ref, w_ref, cbn_ref, sh_ref, gidx_ref):
    xb = x_ref[:]
    xb16 = xb.astype(jnp.bfloat16)
    xsq16 = xb16 * xb16
    ones8 = jnp.ones((8, D), jnp.bfloat16)
    xnorm = lax.dot_general(ones8, xsq16, _DN,
                            preferred_element_type=jnp.float32)[0]
    iota_col = lax.broadcasted_iota(
        jnp.int32, (NCODES, 1), 0).astype(jnp.bfloat16)
    big16 = jnp.bfloat16(256.0)
    sum_s = jnp.zeros((BN,), jnp.float32)
    for m in range(M):
        xc16 = xb16[:, m * DSUB:(m + 1) * DSUB]
        cbn_col = cbn_ref[m][:, None]
        s = cbn_col + lax.dot_general(
            w_ref[m], xc16, _DN,
            preferred_element_type=jnp.float32,
        ).astype(jnp.bfloat16)
        smin = jnp.min(s, axis=0)
        mask = s == smin[None, :]
        idx16 = jnp.min(jnp.where(mask, iota_col, big16), axis=0)
        gidx_ref[m, :] = idx16.astype(jnp.int32) + m * NCODES
        sum_s = sum_s + smin.astype(jnp.float32)
    sh_ref[0, :] = jnp.maximum(xnorm + sum_s, 0.0)
    sh_ref[1, :] = 0.5 * sum_s + xnorm


def _pdt_sc_body(gidx_hbm, sh_hbm, cbnf_hbm, out_hbm,
                 gidx_v, sh_v, cbnf_v, out_v):
    wid = lax.axis_index("s") * 2 + lax.axis_index("c")
    base = wid * RPW
    pltpu.sync_copy(cbnf_hbm, cbnf_v)
    pltpu.sync_copy(gidx_hbm.at[:, pl.ds(base, RPW)], gidx_v)
    pltpu.sync_copy(sh_hbm.at[:, pl.ds(base, RPW)], sh_v)
    half = jnp.full((LANES,), 0.5, jnp.float32)
    tiny = jnp.full((LANES,), 1e-30, jnp.float32)
    magic = jnp.full((LANES,), 0x1FBD1DF5, jnp.int32)
    for i in range(RPW // LANES):
        sl = pl.ds(i * LANES, LANES)
        g = jnp.zeros((LANES,), jnp.float32)
        for m in range(M):
            g = g + plsc.load_gather(cbnf_v, [gidx_v[m, sl]])
        d = jnp.maximum(sh_v[0, sl], tiny)
        y = plsc.bitcast((plsc.bitcast(d, jnp.int32) >> 1) + magic,
                         jnp.float32)
        y = half * (y + d / y)
        y = half * (y + d / y)
        y = half * (y + d / y)
        out_v[sl] = y + jnp.abs(half * g - sh_v[1, sl])
    pltpu.sync_copy(out_v, out_hbm.at[pl.ds(base, RPW)])


def kernel(x, codebook):
    cbn_f32 = jnp.sum(codebook * codebook, axis=-1)
    cbn16 = cbn_f32.astype(jnp.bfloat16)
    w = (-2.0 * codebook).astype(jnp.bfloat16)
    cbnf = cbn_f32.reshape(M * NCODES)

    sh, gidx = pl.pallas_call(
        _pdt_tc_body,
        grid=(N // BN,),
        in_specs=[
            pl.BlockSpec((BN, D), lambda i: (i, 0)),
            pl.BlockSpec((M, NCODES, DSUB), lambda i: (0, 0, 0)),
            pl.BlockSpec((M, NCODES), lambda i: (0, 0)),
        ],
        out_specs=[
            pl.BlockSpec((2, BN), lambda i: (0, i)),
            pl.BlockSpec((M, BN), lambda i: (0, i)),
        ],
        out_shape=[
            jax.ShapeDtypeStruct((2, N), jnp.float32),
            jax.ShapeDtypeStruct((M, N), jnp.int32),
        ],
    )(x, w, cbn16)

    mesh = plsc.VectorSubcoreMesh(core_axis_name="c", subcore_axis_name="s")
    sc_stage = functools.partial(
        pl.kernel,
        mesh=mesh,
        compiler_params=pltpu.CompilerParams(needs_layout_passes=False),
        out_type=jax.ShapeDtypeStruct((N,), jnp.float32),
        scratch_types=[
            pltpu.VMEM((M, RPW), jnp.int32),
            pltpu.VMEM((2, RPW), jnp.float32),
            pltpu.VMEM((M * NCODES,), jnp.float32),
            pltpu.VMEM((RPW,), jnp.float32),
        ],
    )(_pdt_sc_body)
    return sc_stage(gidx, sh, cbnf)
